# Initial kernel scaffold; baseline (speedup 1.0000x reference)
import jax
import jax.numpy as jnp
from jax.experimental import pallas as pl


def kernel(h, x, layer_idx, layer_emb_table, phase_emb_table, router_W1, router_b1, router_ln_g, router_ln_b, router_W2, router_b2, e_W1, e_b1, e_ln_g, e_ln_b, e_W2, e_b2, aW, ab, bW, bb, gW, gb, vW, vb):
    raise NotImplementedError("write your pallas kernel here")


# speedup vs baseline: 2.4247x; 2.4247x over previous
import functools

import jax
import jax.numpy as jnp
from jax import lax
from jax.experimental import pallas as pl
from jax.experimental.pallas import tpu as pltpu
from jax.experimental.pallas import tpu_sc as plsc

N = 2048
D = 1024
E = 8
K = 2
RD = 256
H = 512
TB = 256
NB = 24
CAP = NB * TB
NW = 32
TPW = N // NW


def _lanes_cumsum8(x):
    for k in (1, 2, 4):
        shifted = jnp.concatenate(
            [jnp.zeros(x.shape[:-1] + (k,), x.dtype), x[..., :-k]], axis=-1)
        x = x + shifted
    return x


def _gelu_exact(x):
    return 0.5 * x * (1.0 + lax.erf(x * 0.7071067811865476))


def _sigmoid(x):
    return 1.0 / (1.0 + jnp.exp(-x))


def _softplus(x):
    return jnp.where(x > 20.0, x, jnp.log(1.0 + jnp.exp(jnp.minimum(x, 20.0))))


def _router_body(h_ref, x_ref, a_ref, b_ref, rb_ref, g_ref, bb_ref,
                 w2_ref, b2_ref, oh1_ref, oh2_ref, rw0_ref, rw1_ref):
    f32 = jnp.float32
    z = (jnp.dot(h_ref[...], a_ref[...], preferred_element_type=f32)
         + jnp.dot(x_ref[...], b_ref[...], preferred_element_type=f32)
         + rb_ref[...])
    m = jnp.mean(z, axis=-1, keepdims=True)
    v = jnp.mean((z - m) * (z - m), axis=-1, keepdims=True)
    z = (z - m) * lax.rsqrt(v + 1e-5) * g_ref[...] + bb_ref[...]
    z = _gelu_exact(z)
    logits = jnp.dot(z, w2_ref[...], preferred_element_type=f32) + b2_ref[...]

    m1 = jnp.max(logits, axis=-1, keepdims=True)
    is1 = (logits == m1).astype(f32)
    first1 = jnp.where(_lanes_cumsum8(is1) - is1 < 0.5, is1, 0.0)
    masked = jnp.where(first1 > 0.5, -jnp.inf, logits)
    m2 = jnp.max(masked, axis=-1, keepdims=True)
    is2 = (masked == m2).astype(f32)
    first2 = jnp.where(_lanes_cumsum8(is2) - is2 < 0.5, is2, 0.0)

    rw0 = 1.0 / (1.0 + jnp.exp(m2 - m1))
    rw1 = 1.0 - rw0
    oh1_ref[...] = first1
    oh2_ref[...] = first2
    rw0_ref[...] = jnp.broadcast_to(rw0, rw0_ref.shape)
    rw1_ref[...] = jnp.broadcast_to(rw1, rw1_ref.shape)


def _router(h, x, a, b, rbias, g, bb, w2, b2):
    f32 = jnp.float32
    TN = 256
    return pl.pallas_call(
        _router_body,
        grid=(N // TN,),
        in_specs=[
            pl.BlockSpec((TN, D), lambda i: (i, 0)),
            pl.BlockSpec((TN, D), lambda i: (i, 0)),
            pl.BlockSpec((D, RD), lambda i: (0, 0)),
            pl.BlockSpec((D, RD), lambda i: (0, 0)),
            pl.BlockSpec((1, RD), lambda i: (0, 0)),
            pl.BlockSpec((1, RD), lambda i: (0, 0)),
            pl.BlockSpec((1, RD), lambda i: (0, 0)),
            pl.BlockSpec((RD, E), lambda i: (0, 0)),
            pl.BlockSpec((1, E), lambda i: (0, 0)),
        ],
        out_specs=(
            pl.BlockSpec((TN, E), lambda i: (i, 0)),
            pl.BlockSpec((TN, E), lambda i: (i, 0)),
            pl.BlockSpec((TN, 128), lambda i: (i, 0)),
            pl.BlockSpec((TN, 128), lambda i: (i, 0)),
        ),
        out_shape=(
            jax.ShapeDtypeStruct((N, E), f32),
            jax.ShapeDtypeStruct((N, E), f32),
            jax.ShapeDtypeStruct((N, 128), f32),
            jax.ShapeDtypeStruct((N, 128), f32),
        ),
    )(h, x, a, b, rbias, g, bb, w2, b2)


def _meta_body(oh1_ref, oh2_ref, slot0_ref, slot1_ref, bexp_ref, bval_ref):
    f32 = jnp.float32
    oh1 = oh1_ref[...]
    oh2 = oh2_ref[...]
    both = oh1 + oh2
    inc = both
    k = 1
    while k < N:
        shifted = jnp.concatenate(
            [jnp.zeros((k, E), f32), inc[:-k, :]], axis=0)
        inc = inc + shifted
        k *= 2
    excl = inc - both
    counts = inc[N - 1:N, :]
    padded = jnp.ceil(counts * (1.0 / TB)) * TB
    offs = _lanes_cumsum8(padded) - padded
    rank0 = jnp.sum(excl * oh1, axis=-1, keepdims=True)
    rank1 = jnp.sum((excl + oh1) * oh2, axis=-1, keepdims=True)
    off0 = jnp.sum(offs * oh1, axis=-1, keepdims=True)
    off1 = jnp.sum(offs * oh2, axis=-1, keepdims=True)
    slot0_ref[...] = (off0 + rank0).astype(jnp.int32)
    slot1_ref[...] = (off1 + rank1).astype(jnp.int32)

    total = jnp.sum(padded, axis=-1, keepdims=True)
    bpos = lax.broadcasted_iota(jnp.int32, (NB, E), 0).astype(f32) * TB
    started = (bpos >= jnp.broadcast_to(offs, (NB, E))).astype(f32)
    bexp = jnp.sum(started, axis=-1, keepdims=True) - 1.0
    bval = bpos[:, 0:1] < jnp.broadcast_to(total, (NB, 1))
    bexp_ref[...] = jnp.clip(bexp, 0.0, E - 1.0).astype(jnp.int32)
    bval_ref[...] = bval.astype(jnp.int32)


def _metadata(oh1, oh2):
    i32 = jnp.int32
    return pl.pallas_call(
        _meta_body,
        out_shape=(
            jax.ShapeDtypeStruct((N, 1), i32),
            jax.ShapeDtypeStruct((N, 1), i32),
            jax.ShapeDtypeStruct((NB, 1), i32),
            jax.ShapeDtypeStruct((NB, 1), i32),
        ),
    )(oh1, oh2)


_CHUNK = 8


def _sc_scatter_body(comb_hbm, slot0_hbm, slot1_hbm, xg_hbm,
                     idx0_v, idx1_v, buf, sem0, sem1):
    info = plsc.get_sparse_core_info()
    nc = info.num_cores
    wid = lax.axis_index("s") * nc + lax.axis_index("c")
    base = wid * TPW
    nch = TPW // _CHUNK
    for r in range(nch):
        pltpu.sync_copy(slot0_hbm.at[pl.ds(base + r * _CHUNK, _CHUNK)],
                        idx0_v.at[r])
        pltpu.sync_copy(slot1_hbm.at[pl.ds(base + r * _CHUNK, _CHUNK)],
                        idx1_v.at[r])
    for c in range(nch):
        t0 = base + c * _CHUNK
        pltpu.sync_copy(comb_hbm.at[pl.ds(t0, _CHUNK)], buf)
        h0 = pltpu.async_copy(buf, xg_hbm.at[idx0_v.at[c]], sem0)
        h1 = pltpu.async_copy(buf, xg_hbm.at[idx1_v.at[c]], sem1)
        h0.wait()
        h1.wait()


def _sc_scatter(comb, slot0, slot1):
    nch = TPW // _CHUNK
    mesh = plsc.VectorSubcoreMesh(core_axis_name="c", subcore_axis_name="s")
    f = functools.partial(
        pl.kernel,
        mesh=mesh,
        out_type=jax.ShapeDtypeStruct((CAP, 2 * D), jnp.float32),
        scratch_types=[
            pltpu.VMEM((nch, _CHUNK), jnp.int32),
            pltpu.VMEM((nch, _CHUNK), jnp.int32),
            pltpu.VMEM((_CHUNK, 2 * D), jnp.float32),
            pltpu.SemaphoreType.DMA,
            pltpu.SemaphoreType.DMA,
        ],
    )(_sc_scatter_body)
    return f(comb, slot0, slot1)


def _sc_gather_body(y_hbm, slot0_hbm, slot1_hbm, z0_hbm, z1_hbm,
                    idx0_v, idx1_v, buf0, buf1, sem0, sem1):
    info = plsc.get_sparse_core_info()
    nc = info.num_cores
    wid = lax.axis_index("s") * nc + lax.axis_index("c")
    base = wid * TPW
    nch = TPW // _CHUNK
    for r in range(nch):
        pltpu.sync_copy(slot0_hbm.at[pl.ds(base + r * _CHUNK, _CHUNK)],
                        idx0_v.at[r])
        pltpu.sync_copy(slot1_hbm.at[pl.ds(base + r * _CHUNK, _CHUNK)],
                        idx1_v.at[r])
    for c in range(nch):
        t0 = base + c * _CHUNK
        g0 = pltpu.async_copy(y_hbm.at[idx0_v.at[c]], buf0, sem0)
        g1 = pltpu.async_copy(y_hbm.at[idx1_v.at[c]], buf1, sem1)
        g0.wait()
        pltpu.sync_copy(buf0, z0_hbm.at[pl.ds(t0, _CHUNK)])
        g1.wait()
        pltpu.sync_copy(buf1, z1_hbm.at[pl.ds(t0, _CHUNK)])


def _sc_gather(y, slot0, slot1):
    nch = TPW // _CHUNK
    mesh = plsc.VectorSubcoreMesh(core_axis_name="c", subcore_axis_name="s")
    f = functools.partial(
        pl.kernel,
        mesh=mesh,
        out_type=(
            jax.ShapeDtypeStruct((N, 4 * D), jnp.float32),
            jax.ShapeDtypeStruct((N, 4 * D), jnp.float32),
        ),
        scratch_types=[
            pltpu.VMEM((nch, _CHUNK), jnp.int32),
            pltpu.VMEM((nch, _CHUNK), jnp.int32),
            pltpu.VMEM((_CHUNK, 4 * D), jnp.float32),
            pltpu.VMEM((_CHUNK, 4 * D), jnp.float32),
            pltpu.SemaphoreType.DMA,
            pltpu.SemaphoreType.DMA,
        ],
    )(_sc_gather_body)
    return f(y, slot0, slot1)


def _mlp1_body(bexp_ref, bval_ref, xg_ref, w1_ref, b1_ref, g_ref, b_ref,
               w2_ref, b2_ref, o_ref):
    i = pl.program_id(0)

    @pl.when(bval_ref[i] == 1)
    def _():
        f32 = jnp.float32
        t = (jnp.dot(xg_ref[...], w1_ref[0], preferred_element_type=f32)
             + b1_ref[0])
        m = jnp.mean(t, axis=-1, keepdims=True)
        v = jnp.mean((t - m) * (t - m), axis=-1, keepdims=True)
        t = (t - m) * lax.rsqrt(v + 1e-5) * g_ref[0] + b_ref[0]
        t = _gelu_exact(t)
        o_ref[...] = (jnp.dot(t, w2_ref[0], preferred_element_type=f32)
                      + b2_ref[0])


def _mlp2_body(bexp_ref, bval_ref, o_ref, aw_ref, ab_ref, bw_ref, bb_ref,
               gw_ref, gb_ref, vw_ref, vb_ref, y_ref):
    i = pl.program_id(0)

    @pl.when(bval_ref[i] == 1)
    def _():
        f32 = jnp.float32
        o = o_ref[...]
        a = (jnp.dot(o[:, 0 * D:1 * D], aw_ref[0], preferred_element_type=f32)
             + ab_ref[0])
        b = (jnp.dot(o[:, 1 * D:2 * D], bw_ref[0], preferred_element_type=f32)
             + bb_ref[0])
        g = (jnp.dot(o[:, 2 * D:3 * D], gw_ref[0], preferred_element_type=f32)
             + gb_ref[0])
        v = (jnp.dot(o[:, 3 * D:4 * D], vw_ref[0], preferred_element_type=f32)
             + vb_ref[0])
        y_ref[:, 0 * D:1 * D] = _sigmoid(a)
        y_ref[:, 1 * D:2 * D] = _softplus(b)
        y_ref[:, 2 * D:3 * D] = _sigmoid(g)
        y_ref[:, 3 * D:4 * D] = v


def _grouped_mlp(xg, bexp, bval, e_W1, e_b1, e_ln_g, e_ln_b, e_W2, e_b2,
                 aW, ab, bW, bb, gW, gb, vW, vb):
    f32 = jnp.float32

    def wmap(i, bexp, bval):
        return (bexp[i], 0, 0)

    def bmap(i, bexp, bval):
        return (bexp[i], 0, 0)

    def xmap(i, bexp, bval):
        return (i, 0)

    o = pl.pallas_call(
        _mlp1_body,
        grid_spec=pltpu.PrefetchScalarGridSpec(
            num_scalar_prefetch=2,
            grid=(NB,),
            in_specs=[
                pl.BlockSpec((TB, 2 * D), xmap),
                pl.BlockSpec((1, 2 * D, H), wmap),
                pl.BlockSpec((1, 1, H), bmap),
                pl.BlockSpec((1, 1, H), bmap),
                pl.BlockSpec((1, 1, H), bmap),
                pl.BlockSpec((1, H, 4 * D), wmap),
                pl.BlockSpec((1, 1, 4 * D), bmap),
            ],
            out_specs=pl.BlockSpec((TB, 4 * D), xmap),
        ),
        out_shape=jax.ShapeDtypeStruct((CAP, 4 * D), f32),
    )(bexp, bval, xg, e_W1, e_b1, e_ln_g, e_ln_b, e_W2, e_b2)

    y = pl.pallas_call(
        _mlp2_body,
        grid_spec=pltpu.PrefetchScalarGridSpec(
            num_scalar_prefetch=2,
            grid=(NB,),
            in_specs=[
                pl.BlockSpec((TB, 4 * D), xmap),
                pl.BlockSpec((1, D, D), wmap),
                pl.BlockSpec((1, 1, D), bmap),
                pl.BlockSpec((1, D, D), wmap),
                pl.BlockSpec((1, 1, D), bmap),
                pl.BlockSpec((1, D, D), wmap),
                pl.BlockSpec((1, 1, D), bmap),
                pl.BlockSpec((1, D, D), wmap),
                pl.BlockSpec((1, 1, D), bmap),
            ],
            out_specs=pl.BlockSpec((TB, 4 * D), xmap),
        ),
        out_shape=jax.ShapeDtypeStruct((CAP, 4 * D), f32),
    )(bexp, bval, o, aW, ab, bW, bb, gW, gb, vW, vb)
    return y


def _combine_body(z0_ref, z1_ref, rw0_ref, rw1_ref, out_ref):
    w0 = rw0_ref[:, 0:1]
    w1 = rw1_ref[:, 0:1]
    out_ref[...] = z0_ref[...] * w0 + z1_ref[...] * w1


def _combine(z0, z1, rw0b, rw1b):
    TN = 256
    return pl.pallas_call(
        _combine_body,
        grid=(N // TN,),
        in_specs=[
            pl.BlockSpec((TN, 4 * D), lambda i: (i, 0)),
            pl.BlockSpec((TN, 4 * D), lambda i: (i, 0)),
            pl.BlockSpec((TN, 128), lambda i: (i, 0)),
            pl.BlockSpec((TN, 128), lambda i: (i, 0)),
        ],
        out_specs=pl.BlockSpec((TN, 4 * D), lambda i: (i, 0)),
        out_shape=jax.ShapeDtypeStruct((N, 4 * D), jnp.float32),
    )(z0, z1, rw0b, rw1b)


def kernel(h, x, layer_idx, layer_emb_table, phase_emb_table, router_W1,
           router_b1, router_ln_g, router_ln_b, router_W2, router_b2,
           e_W1, e_b1, e_ln_g, e_ln_b, e_W2, e_b2,
           aW, ab, bW, bb, gW, gb, vW, vb):
    le = lax.dynamic_slice_in_dim(layer_emb_table, layer_idx, 1, 0)
    pe = phase_emb_table[0:1]
    a = router_W1[:D]
    b = router_W1[D:2 * D]
    rbias = (le @ router_W1[2 * D:2 * D + 32]
             + pe @ router_W1[2 * D + 32:]
             + router_b1.reshape(1, RD))

    oh1, oh2, rw0b, rw1b = _router(
        h, x, a, b, rbias, router_ln_g.reshape(1, RD),
        router_ln_b.reshape(1, RD), router_W2, router_b2.reshape(1, E))

    slot0, slot1, bexp, bval = _metadata(oh1, oh2)
    slot0 = slot0.reshape(N)
    slot1 = slot1.reshape(N)
    bexp = bexp.reshape(NB)
    bval = bval.reshape(NB)

    comb = jnp.concatenate([h, x], axis=-1)
    xg = _sc_scatter(comb, slot0, slot1)

    y = _grouped_mlp(xg, bexp, bval, e_W1, e_b1.reshape(E, 1, H),
                     e_ln_g.reshape(E, 1, H), e_ln_b.reshape(E, 1, H), e_W2,
                     e_b2.reshape(E, 1, 4 * D), aW, ab.reshape(E, 1, D),
                     bW, bb.reshape(E, 1, D), gW, gb.reshape(E, 1, D),
                     vW, vb.reshape(E, 1, D))

    z0, z1 = _sc_gather(y, slot0, slot1)

    out = _combine(z0, z1, rw0b, rw1b)
    alpha = out[:, 0 * D:1 * D]
    beta = out[:, 1 * D:2 * D]
    gate = out[:, 2 * D:3 * D]
    v_cand = out[:, 3 * D:4 * D]
    return (alpha, beta, gate, v_cand)
